# Initial kernel scaffold; baseline (speedup 1.0000x reference)
import jax
import jax.numpy as jnp
from jax.experimental import pallas as pl


def kernel(x, edge_idx, W_embed, b_embed, w_ih, w_hh, b_ih, b_hh, Wh, bh, Wo, bo):
    raise NotImplementedError("write your pallas kernel here")


# speedup vs baseline: 6.1137x; 6.1137x over previous
import functools

import jax
import jax.numpy as jnp
from jax import lax
from jax.experimental import pallas as pl
from jax.experimental.pallas import tpu as pltpu
from jax.experimental.pallas import tpu_sc as plsc

N = 10000
E = 320000
D = 128
NC = 2
NS = 16
NW = NC * NS
CHUNK = 128
DUMMY = N
HALF0 = 5056
ACC_ROWS = 5120
DUMMY_A = ACC_ROWS - 1
RPT = ACC_ROWS // NS
DEG_ROWS = 16384
DPT = DEG_ROWS // NS
EPW = E // NW
NCHUNK_D = -(-EPW // CHUNK)
EPW_PAD = NCHUNK_D * CHUNK
EPT = E // NS
NCHUNK_A = -(-EPT // CHUNK)
EPT_PAD = NCHUNK_A * CHUNK


def _deg_body(cols_hbm, out_hbm, col_v, ones_v, zero_v, acc):
    c = lax.axis_index("c")
    s = lax.axis_index("s")
    w = c * NS + s
    pltpu.sync_copy(cols_hbm.at[w], col_v)
    for j in range(CHUNK // 16):
        ones_v[pl.ds(j * 16, 16)] = jnp.ones((16,), jnp.float32)

    def zrow(i, _):
        zero_v[pl.ds(i * 16, 16)] = jnp.zeros((16,), jnp.float32)
        return 0

    lax.fori_loop(0, DPT // 16, zrow, 0)
    pltpu.sync_copy(zero_v, acc.at[pl.ds(s * DPT, DPT)])
    plsc.subcore_barrier()

    def body(j, _):
        pltpu.sync_copy(ones_v, acc.at[col_v.at[j]], add=True)
        return 0

    lax.fori_loop(0, NCHUNK_D, body, 0)
    plsc.subcore_barrier()
    pltpu.sync_copy(acc.at[pl.ds(s * DPT, DPT)],
                    out_hbm.at[pl.ds(c * DEG_ROWS + s * DPT, DPT)])


def _agg_body(g_hbm, rows_hbm, cols_hbm, out_hbm, row_v, col_v, buf, acc,
              sem0, sem1):
    c = lax.axis_index("c")
    s = lax.axis_index("s")
    pltpu.sync_copy(rows_hbm.at[s], row_v)
    pltpu.sync_copy(cols_hbm.at[s], col_v)

    lo = c * HALF0

    def remap(i, _):
        for j in range(CHUNK // 16):
            v = col_v[i, pl.ds(j * 16, 16)] - lo
            keep = (v >= 0) & (v < HALF0)
            col_v[i, pl.ds(j * 16, 16)] = jnp.where(keep, v, DUMMY_A)
        return 0

    lax.fori_loop(0, NCHUNK_A, remap, 0)

    def zrow(i, _):
        for j in range(D // 16):
            buf[0, i, pl.ds(j * 16, 16)] = jnp.zeros((16,), jnp.float32)
        return 0

    lax.fori_loop(0, CHUNK, zrow, 0)
    base = s * RPT
    off = 0
    while off < RPT:
        n = min(CHUNK, RPT - off)
        pltpu.sync_copy(buf.at[0, pl.ds(0, n), :], acc.at[pl.ds(base + off, n), :])
        off += n
    plsc.subcore_barrier()

    pltpu.async_copy(g_hbm.at[row_v.at[0]], buf.at[0], sem0)

    def body(t, _):
        j0 = 2 * t
        pltpu.async_copy(g_hbm.at[row_v.at[j0 + 1]], buf.at[1], sem1)
        pltpu.make_async_copy(g_hbm.at[row_v.at[j0]], buf.at[0], sem0).wait()
        pltpu.sync_copy(buf.at[0], acc.at[col_v.at[j0]], add=True)
        pltpu.async_copy(g_hbm.at[row_v.at[j0 + 2]], buf.at[0], sem0)
        pltpu.make_async_copy(g_hbm.at[row_v.at[j0 + 1]], buf.at[1], sem1).wait()
        pltpu.sync_copy(buf.at[1], acc.at[col_v.at[j0 + 1]], add=True)
        return 0

    lax.fori_loop(0, (NCHUNK_A - 1) // 2, body, 0)
    pltpu.make_async_copy(g_hbm.at[row_v.at[NCHUNK_A - 1]], buf.at[0], sem0).wait()
    pltpu.sync_copy(buf.at[0], acc.at[col_v.at[NCHUNK_A - 1]], add=True)
    plsc.subcore_barrier()
    pltpu.sync_copy(acc.at[pl.ds(base, RPT), :],
                    out_hbm.at[c, pl.ds(base, RPT), :])


@functools.cache
def _sc_kernels():
    mesh = plsc.VectorSubcoreMesh(core_axis_name="c", subcore_axis_name="s",
                                  num_cores=NC, num_subcores=NS)
    deg = pl.kernel(
        _deg_body,
        out_type=jax.ShapeDtypeStruct((NC * DEG_ROWS,), jnp.float32),
        mesh=mesh,
        scratch_types=[
            pltpu.VMEM((NCHUNK_D, CHUNK), jnp.int32),
            pltpu.VMEM((CHUNK,), jnp.float32),
            pltpu.VMEM((DPT,), jnp.float32),
            pltpu.VMEM_SHARED((DEG_ROWS,), jnp.float32),
        ],
    )
    agg = pl.kernel(
        _agg_body,
        out_type=jax.ShapeDtypeStruct((NC, ACC_ROWS, D), jnp.float32),
        mesh=mesh,
        scratch_types=[
            pltpu.VMEM((NCHUNK_A, CHUNK), jnp.int32),
            pltpu.VMEM((NCHUNK_A, CHUNK), jnp.int32),
            pltpu.VMEM((2, CHUNK, D), jnp.float32),
            pltpu.VMEM_SHARED((ACC_ROWS, D), jnp.float32),
            pltpu.SemaphoreType.DMA,
            pltpu.SemaphoreType.DMA,
        ],
    )
    return deg, agg


_BR = 1000
_GRID = N // _BR


def _mm_t(a, w):
    return lax.dot_general(a, w, (((1,), (1,)), ((), ())),
                           preferred_element_type=jnp.float32)


def _enc_body(xp_ref, w_ref, b_ref, dga_ref, dgb_ref, h_ref, g_ref, d_ref):
    h = jnp.maximum(_mm_t(xp_ref[...], w_ref[...]) + b_ref[...], 0.0)
    nrm = jnp.sqrt(jnp.sum(h * h, axis=1, keepdims=True))
    h = h / jnp.maximum(nrm, 1e-12)
    d = lax.rsqrt(dga_ref[...] + dgb_ref[...] + 1.0)
    h_ref[...] = h
    g_ref[...] = h * d
    d_ref[...] = d


def _gru_math(s_ref, h_ref, d_ref, wih_ref, whh_ref, bih_ref, bhh_ref):
    h = h_ref[...]
    agg = d_ref[...] * s_ref[...]
    gi = _mm_t(agg, wih_ref[...]) + bih_ref[...]
    gh = _mm_t(h, whh_ref[...]) + bhh_ref[...]
    r = jax.nn.sigmoid(gi[:, :D] + gh[:, :D])
    z = jax.nn.sigmoid(gi[:, D:2 * D] + gh[:, D:2 * D])
    n = jnp.tanh(gi[:, 2 * D:] + r * gh[:, 2 * D:])
    hn = (1.0 - z) * n + z * h
    nrm = jnp.sqrt(jnp.sum(hn * hn, axis=1, keepdims=True))
    return hn / jnp.maximum(nrm, 1e-12)


def _gru_body(s_ref, h_ref, zm_ref, d_ref, wih_ref, whh_ref,
              bih_ref, bhh_ref, hn_ref, gn_ref, zmn_ref):
    hn = _gru_math(s_ref, h_ref, d_ref, wih_ref, whh_ref, bih_ref, bhh_ref)
    hn_ref[...] = hn
    gn_ref[...] = hn * d_ref[...]
    zmn_ref[...] = jnp.maximum(zm_ref[...], hn)


def _gru_final_body(s_ref, h_ref, zm_ref, d_ref, wih_ref, whh_ref,
                    bih_ref, bhh_ref, wh_ref, bh_ref, wo_ref, bo_ref, out_ref):
    hn = _gru_math(s_ref, h_ref, d_ref, wih_ref, whh_ref, bih_ref, bhh_ref)
    zm = jnp.maximum(zm_ref[...], hn)
    hid = jnp.maximum(_mm_t(zm, wh_ref[...]) + bh_ref[...], 0.0)
    out_ref[...] = jnp.sum(hid * wo_ref[...], axis=1, keepdims=True) + bo_ref[0, 0]


def _rows(block_cols):
    return pl.BlockSpec((_BR, block_cols), lambda i: (i, 0))


def _full(shape):
    return pl.BlockSpec(shape, lambda i: tuple(0 for _ in shape))


def _enc_call(xp, wp, b, dga, dgb):
    return pl.pallas_call(
        _enc_body,
        grid=(_GRID,),
        in_specs=[_rows(D), _full((D, D)), _full((1, D)), _rows(1), _rows(1)],
        out_specs=[_rows(D), _rows(D), _rows(1)],
        out_shape=[
            jax.ShapeDtypeStruct((N, D), jnp.float32),
            jax.ShapeDtypeStruct((N, D), jnp.float32),
            jax.ShapeDtypeStruct((N, 1), jnp.float32),
        ],
    )(xp, wp, b, dga, dgb)


def _gru_call(s, h, zm, d, wih, whh, bih, bhh):
    return pl.pallas_call(
        _gru_body,
        grid=(_GRID,),
        in_specs=[_rows(D), _rows(D), _rows(D), _rows(1),
                  _full((3 * D, D)), _full((3 * D, D)),
                  _full((1, 3 * D)), _full((1, 3 * D))],
        out_specs=[_rows(D), _rows(D), _rows(D)],
        out_shape=[
            jax.ShapeDtypeStruct((N, D), jnp.float32),
            jax.ShapeDtypeStruct((N, D), jnp.float32),
            jax.ShapeDtypeStruct((N, D), jnp.float32),
        ],
    )(s, h, zm, d, wih, whh, bih, bhh)


def _gru_final_call(s, h, zm, d, wih, whh, bih, bhh, wh, bh, wo, bo):
    return pl.pallas_call(
        _gru_final_body,
        grid=(_GRID,),
        in_specs=[_rows(D), _rows(D), _rows(D), _rows(1),
                  _full((3 * D, D)), _full((3 * D, D)),
                  _full((1, 3 * D)), _full((1, 3 * D)),
                  _full((64, D)), _full((1, 64)), _full((1, 64)), _full((1, 1))],
        out_specs=[_rows(1)],
        out_shape=[jax.ShapeDtypeStruct((N, 1), jnp.float32)],
    )(s, h, zm, d, wih, whh, bih, bhh, wh, bh, wo, bo)


def kernel(x, edge_idx, W_embed, b_embed, w_ih, w_hh, b_ih, b_hh, Wh, bh, Wo, bo):
    row = edge_idx[0].astype(jnp.int32)
    col = edge_idx[1].astype(jnp.int32)
    col_w = jnp.pad(col.reshape(NW, EPW), ((0, 0), (0, EPW_PAD - EPW)),
                    constant_values=DUMMY).reshape(NW, NCHUNK_D, CHUNK)
    row_t = jnp.pad(row.reshape(NS, EPT), ((0, 0), (0, EPT_PAD - EPT))
                    ).reshape(NS, NCHUNK_A, CHUNK)
    col_t = jnp.pad(col.reshape(NS, EPT), ((0, 0), (0, EPT_PAD - EPT)),
                    constant_values=DUMMY).reshape(NS, NCHUNK_A, CHUNK)

    deg_kernel, agg_kernel = _sc_kernels()
    deg2 = deg_kernel(col_w)
    dga = deg2[:N].reshape(N, 1)
    dgb = deg2[DEG_ROWS:DEG_ROWS + N].reshape(N, 1)

    xp = jnp.pad(x, ((0, 0), (0, D - x.shape[1])))
    wp = jnp.pad(W_embed, ((0, 0), (0, D - W_embed.shape[1])))
    h, g, d = _enc_call(xp, wp, b_embed.reshape(1, D), dga, dgb)
    zm = h
    out = None
    for i in range(5):
        S = agg_kernel(g, row_t, col_t)
        s = jnp.concatenate([S[0, :HALF0], S[1, :N - HALF0]], axis=0)
        bih = b_ih[i].reshape(1, 3 * D)
        bhh = b_hh[i].reshape(1, 3 * D)
        if i < 4:
            h, g, zm = _gru_call(s, h, zm, d, w_ih[i], w_hh[i], bih, bhh)
        else:
            out, = _gru_final_call(s, h, zm, d, w_ih[i], w_hh[i], bih, bhh,
                                   Wh, bh.reshape(1, 64), Wo, bo.reshape(1, 1))
    return out
